# baseline (speedup 1.0000x reference)
import functools

import jax
import jax.numpy as jnp
from jax import lax
from jax.experimental import pallas as pl
from jax.experimental.pallas import tpu as pltpu
from jax.experimental.pallas import tpu_sc as plsc

D = 256
L = 16
NC, NS = 2, 16
NW = NC * NS
NPAD = 10240
EPAD = 163840
EW = EPAD // NW
EW0 = 7424
EW1 = EPAD // NS - EW0
CH1 = 64
CH2 = 128
ESC = EPAD // NS
NCH2 = ESC // CH2
RPT = NPAD // NS
MBLK = 1024


def _rsqrt_v(v):
    i = plsc.bitcast(v, jnp.int32)
    y = plsc.bitcast(jnp.int32(0x5F3759DF) - lax.shift_right_arithmetic(i, 1),
                     jnp.float32)
    for _ in range(3):
        y = y * (1.5 - 0.5 * v * y * y)
    return y


_sc_mesh = plsc.VectorSubcoreMesh(core_axis_name="c", subcore_axis_name="s")
_sc_params = pltpu.CompilerParams(needs_layout_passes=False)


@functools.partial(
    pl.kernel,
    out_type=jax.ShapeDtypeStruct((2 * EPAD, 128), jnp.float32),
    mesh=_sc_mesh,
    compiler_params=_sc_params,
    scratch_types=[
        pltpu.VMEM((2, CH1), jnp.int32),
        pltpu.VMEM((2, CH1), jnp.int32),
        pltpu.VMEM((2, CH1, D // 2), jnp.uint32),
        pltpu.VMEM((2, CH1, D // 2), jnp.uint32),
        pltpu.VMEM((2, CH1, 128), jnp.float32),
        pltpu.VMEM((2, CH1, 128), jnp.float32),
        pltpu.SemaphoreType.DMA,
        pltpu.SemaphoreType.DMA,
        pltpu.SemaphoreType.DMA,
        pltpu.SemaphoreType.DMA,
        pltpu.SemaphoreType.DMA,
        pltpu.SemaphoreType.DMA,
    ],
)
def _msg_kernel(p_hbm, q_hbm, r_hbm, s_hbm, out_hbm,
                ridx, sidx, pbuf, qbuf, mlo, mhi,
                sp0, sp1, sq0, sq1, so0, so1):
    cid = lax.axis_index("c")
    sid = lax.axis_index("s")
    is0 = cid == 0
    tcnt = jnp.where(is0, EW0, EW1)
    e0 = jnp.where(is0, sid * EW0, NS * EW0 + sid * EW1)
    nch = tcnt // CH1
    semp = [sp0, sp1]
    semq = [sq0, sq1]
    semo = [so0, so1]

    def fire(b, i):
        base = e0 + i * CH1
        pltpu.sync_copy(r_hbm.at[pl.ds(base, CH1)], ridx.at[b])
        pltpu.sync_copy(s_hbm.at[pl.ds(base, CH1)], sidx.at[b])
        pltpu.async_copy(p_hbm.at[ridx.at[b]], pbuf.at[b], semp[b])
        pltpu.async_copy(q_hbm.at[sidx.at[b]], qbuf.at[b], semq[b])

    fire(0, 0)

    def pair_body(i2, carry):
        for b in range(2):
            i = 2 * i2 + b
            base = e0 + i * CH1
            pltpu.make_async_copy(
                p_hbm.at[ridx.at[b]], pbuf.at[b], semp[b]).wait()
            pltpu.make_async_copy(
                q_hbm.at[sidx.at[b]], qbuf.at[b], semq[b]).wait()
            nxt = i + 1

            @pl.when(nxt < nch)
            def _():
                fire(1 - b, nxt)

            @pl.when(i >= 2)
            def _():
                pltpu.make_async_copy(
                    mlo.at[b], out_hbm.at[pl.ds(e0, CH1)], semo[b]).wait()
                pltpu.make_async_copy(
                    mhi.at[b], out_hbm.at[pl.ds(e0, CH1)], semo[b]).wait()

            @plsc.parallel_loop(0, CH1, unroll=2)
            def edge_body(j):
                acc1 = jnp.zeros((L,), jnp.float32)
                acc2 = jnp.zeros((L,), jnp.float32)
                xs = []
                for k in range(D // (2 * L)):
                    sl = pl.ds(k * L, L)
                    pb16 = plsc.bitcast(pbuf[b, j, sl], jnp.bfloat16)
                    qb16 = plsc.bitcast(qbuf[b, j, sl], jnp.bfloat16)
                    xb = pb16 + qb16
                    xe, xo = plsc.unpack(xb, format=plsc.PackFormat.INTERLEAVED)
                    xs.append(xe)
                    xs.append(xo)
                    acc1 = acc1 + xe + xo
                    acc2 = acc2 + xe * xe + xo * xo
                s1 = jnp.sum(acc1)
                s2 = jnp.sum(acc2)
                mu = s1 * (1.0 / D)
                var = s2 * (1.0 / D) - mu * mu
                rs = _rsqrt_v(jnp.full((L,), 1e-5, jnp.float32) + var)
                vmu = jnp.zeros((L,), jnp.float32) + mu
                one = jnp.full((L,), 1.0, jnp.float32)
                for k in range(D // (2 * L)):
                    sl = pl.ds(k * L, L)
                    mlo[b, j, sl] = (xs[2 * k] - vmu) * rs + one
                    mhi[b, j, sl] = (xs[2 * k + 1] - vmu) * rs + one

            pltpu.async_copy(mlo.at[b], out_hbm.at[pl.ds(base, CH1)], semo[b])
            pltpu.async_copy(mhi.at[b], out_hbm.at[pl.ds(EPAD + base, CH1)],
                             semo[b])
        return carry

    lax.fori_loop(0, nch // 2, pair_body, 0)
    for b in range(2):
        pltpu.make_async_copy(
            mlo.at[b], out_hbm.at[pl.ds(e0, CH1)], semo[b]).wait()
        pltpu.make_async_copy(
            mhi.at[b], out_hbm.at[pl.ds(e0, CH1)], semo[b]).wait()


@functools.partial(
    pl.kernel,
    out_type=jax.ShapeDtypeStruct((2 * NPAD, 128), jnp.float32),
    mesh=_sc_mesh,
    compiler_params=_sc_params,
    scratch_types=[
        pltpu.VMEM((2, CH2), jnp.int32),
        pltpu.VMEM((2, CH2, 128), jnp.float32),
        pltpu.VMEM_SHARED((NPAD, 128), jnp.float32),
        pltpu.SemaphoreType.DMA,
        pltpu.SemaphoreType.DMA,
    ],
)
def _scatter_kernel(m_hbm, r_hbm, out_hbm, ridx, chunk, acc, sm0, sm1):
    cid = lax.axis_index("c")
    sid = lax.axis_index("s")
    semm = [sm0, sm1]

    def zrow(j, c2):
        for k in range(128 // L):
            chunk[0, j, pl.ds(k * L, L)] = jnp.zeros((L,), jnp.float32)
        return c2

    lax.fori_loop(0, CH2, zrow, 0)
    for m in range(RPT // CH2):
        pltpu.sync_copy(chunk.at[0], acc.at[pl.ds(sid * RPT + m * CH2, CH2)])
    plsc.subcore_barrier()

    def fire(b, i):
        base = sid * ESC + i * CH2
        pltpu.sync_copy(r_hbm.at[pl.ds(base, CH2)], ridx.at[b])
        pltpu.async_copy(m_hbm.at[pl.ds(cid * EPAD + base, CH2)],
                         chunk.at[b], semm[b])

    fire(0, 0)

    def pair_body(i2, carry):
        for b in range(2):
            i = 2 * i2 + b
            pltpu.make_async_copy(
                m_hbm.at[pl.ds(cid * EPAD, CH2)], chunk.at[b],
                semm[b]).wait()
            nxt = i + 1

            @pl.when(nxt < NCH2)
            def _():
                fire(1 - b, nxt)

            pltpu.sync_copy(chunk.at[b], acc.at[ridx.at[b]], add=True)
        return carry

    lax.fori_loop(0, NCH2 // 2, pair_body, 0)
    plsc.subcore_barrier()
    rb = sid * RPT
    pltpu.sync_copy(acc.at[pl.ds(rb, RPT)],
                    out_hbm.at[pl.ds(cid * NPAD + rb, RPT)])


def _proj_body(x_ref, wt_ref, wb_ref, bm_ref, p_ref, q_ref):
    def pack_halves(v):
        lo = lax.bitcast_convert_type(
            v[:, :D // 2].astype(jnp.bfloat16), jnp.uint16).astype(jnp.uint32)
        hi = lax.bitcast_convert_type(
            v[:, D // 2:].astype(jnp.bfloat16), jnp.uint16).astype(jnp.uint32)
        return lo | (hi << 16)

    pv = (jnp.dot(x_ref[...], wt_ref[...],
                  preferred_element_type=jnp.float32) + bm_ref[...])
    qv = jnp.dot(x_ref[...], wb_ref[...], preferred_element_type=jnp.float32)
    p_ref[...] = pack_halves(pv)
    q_ref[...] = pack_halves(qv)


def _update_body(x_ref, lo_ref, hi_ref, w1_ref, w2a_ref, w2b_ref,
                 g1_ref, be1_ref, b_ref, g_ref, be_ref, o_ref):
    g1v = g1_ref[...]
    w2a = w2a_ref[...]
    w2b = w2b_ref[...]
    lo = lo_ref[...]
    hi = hi_ref[...]
    cnt = (jnp.sum(lo, axis=-1, keepdims=True)
           + jnp.sum(hi, axis=-1, keepdims=True)) * (1.0 / D)
    acc = jnp.dot(x_ref[...], w1_ref[...], preferred_element_type=jnp.float32)
    acc = acc + jnp.dot((lo - cnt) * g1v[0, :128], w2a,
                        preferred_element_type=jnp.float32)
    acc = acc + jnp.dot((hi - cnt) * g1v[0, 128:], w2b,
                        preferred_element_type=jnp.float32)
    be1v = be1_ref[...]
    bev = jnp.dot(be1v[:, :128], w2a, preferred_element_type=jnp.float32)
    bev = bev + jnp.dot(be1v[:, 128:], w2b, preferred_element_type=jnp.float32)
    acc = acc + b_ref[...] + cnt * bev
    mu = jnp.mean(acc, axis=-1, keepdims=True)
    var = jnp.mean((acc - mu) ** 2, axis=-1, keepdims=True)
    o_ref[...] = (acc - mu) * lax.rsqrt(var + 1e-5) * g_ref[...] + be_ref[...]


def kernel(nodes, senders, receivers, W_msg, b_msg, g1, be1,
           W_node, b_node, g2, be2):
    n = nodes.shape[1]
    e = senders.shape[0]
    x = jnp.pad(nodes[0], ((0, NPAD - n), (0, 0)))
    rp = jnp.concatenate(
        [receivers, jnp.full((EPAD - e,), n, jnp.int32)])
    sp = jnp.concatenate(
        [senders, jnp.zeros((EPAD - e,), jnp.int32)])

    grid = NPAD // MBLK
    p, q = pl.pallas_call(
        _proj_body,
        grid=(grid,),
        in_specs=[
            pl.BlockSpec((MBLK, D), lambda i: (i, 0)),
            pl.BlockSpec((D, D), lambda i: (0, 0)),
            pl.BlockSpec((D, D), lambda i: (0, 0)),
            pl.BlockSpec((1, D), lambda i: (0, 0)),
        ],
        out_specs=[
            pl.BlockSpec((MBLK, D // 2), lambda i: (i, 0)),
            pl.BlockSpec((MBLK, D // 2), lambda i: (i, 0)),
        ],
        out_shape=[
            jax.ShapeDtypeStruct((NPAD, D // 2), jnp.uint32),
            jax.ShapeDtypeStruct((NPAD, D // 2), jnp.uint32),
        ],
    )(x, W_msg[:D], W_msg[D:], b_msg[None])

    msgs = _msg_kernel(p, q, rp, sp)
    inbox2 = _scatter_kernel(msgs, rp)

    out = pl.pallas_call(
        _update_body,
        grid=(grid,),
        in_specs=[
            pl.BlockSpec((MBLK, D), lambda i: (i, 0)),
            pl.BlockSpec((MBLK, 128), lambda i: (i, 0)),
            pl.BlockSpec((MBLK, 128), lambda i: (i + NPAD // MBLK, 0)),
            pl.BlockSpec((D, D), lambda i: (0, 0)),
            pl.BlockSpec((128, D), lambda i: (0, 0)),
            pl.BlockSpec((128, D), lambda i: (0, 0)),
            pl.BlockSpec((1, D), lambda i: (0, 0)),
            pl.BlockSpec((1, D), lambda i: (0, 0)),
            pl.BlockSpec((1, D), lambda i: (0, 0)),
            pl.BlockSpec((1, D), lambda i: (0, 0)),
            pl.BlockSpec((1, D), lambda i: (0, 0)),
        ],
        out_specs=pl.BlockSpec((MBLK, D), lambda i: (i, 0)),
        out_shape=jax.ShapeDtypeStruct((NPAD, D), jnp.float32),
    )(x, inbox2, inbox2, W_node[:D], W_node[D:D + 128], W_node[D + 128:],
      g1[None], be1[None], b_node[None], g2[None], be2[None])
    return out[:n][None]


# speedup vs baseline: 1.0093x; 1.0093x over previous
import functools

import jax
import jax.numpy as jnp
from jax import lax
from jax.experimental import pallas as pl
from jax.experimental.pallas import tpu as pltpu
from jax.experimental.pallas import tpu_sc as plsc

D = 256
L = 16
NC, NS = 2, 16
NW = NC * NS
NPAD = 10240
EPAD = 163840
EW = EPAD // NW
EW0 = 6912
EW1 = EPAD // NS - EW0
CH1 = 64
CH2 = 128
ESC = EPAD // NS
NCH2 = ESC // CH2
RPT = NPAD // NS
MBLK = 1024


def _rsqrt_v(v):
    i = plsc.bitcast(v, jnp.int32)
    y = plsc.bitcast(jnp.int32(0x5F3759DF) - lax.shift_right_arithmetic(i, 1),
                     jnp.float32)
    for _ in range(3):
        y = y * (1.5 - 0.5 * v * y * y)
    return y


_sc_mesh = plsc.VectorSubcoreMesh(core_axis_name="c", subcore_axis_name="s")
_sc_params = pltpu.CompilerParams(needs_layout_passes=False)


@functools.partial(
    pl.kernel,
    out_type=jax.ShapeDtypeStruct((2 * EPAD, 128), jnp.float32),
    mesh=_sc_mesh,
    compiler_params=_sc_params,
    scratch_types=[
        pltpu.VMEM((2, CH1), jnp.int32),
        pltpu.VMEM((2, CH1), jnp.int32),
        pltpu.VMEM((2, CH1, D // 2), jnp.uint32),
        pltpu.VMEM((2, CH1, D // 2), jnp.uint32),
        pltpu.VMEM((2, CH1, 128), jnp.float32),
        pltpu.VMEM((2, CH1, 128), jnp.float32),
        pltpu.SemaphoreType.DMA,
        pltpu.SemaphoreType.DMA,
        pltpu.SemaphoreType.DMA,
        pltpu.SemaphoreType.DMA,
        pltpu.SemaphoreType.DMA,
        pltpu.SemaphoreType.DMA,
    ],
)
def _msg_kernel(p_hbm, q_hbm, r_hbm, s_hbm, out_hbm,
                ridx, sidx, pbuf, qbuf, mlo, mhi,
                sp0, sp1, sq0, sq1, so0, so1):
    cid = lax.axis_index("c")
    sid = lax.axis_index("s")
    is0 = cid == 0
    tcnt = jnp.where(is0, EW0, EW1)
    e0 = jnp.where(is0, sid * EW0, NS * EW0 + sid * EW1)
    nch = tcnt // CH1
    semp = [sp0, sp1]
    semq = [sq0, sq1]
    semo = [so0, so1]

    def fire(b, i):
        base = e0 + i * CH1
        pltpu.sync_copy(r_hbm.at[pl.ds(base, CH1)], ridx.at[b])
        pltpu.sync_copy(s_hbm.at[pl.ds(base, CH1)], sidx.at[b])
        pltpu.async_copy(p_hbm.at[ridx.at[b]], pbuf.at[b], semp[b])
        pltpu.async_copy(q_hbm.at[sidx.at[b]], qbuf.at[b], semq[b])

    fire(0, 0)

    def pair_body(i2, carry):
        for b in range(2):
            i = 2 * i2 + b
            base = e0 + i * CH1
            pltpu.make_async_copy(
                p_hbm.at[ridx.at[b]], pbuf.at[b], semp[b]).wait()
            pltpu.make_async_copy(
                q_hbm.at[sidx.at[b]], qbuf.at[b], semq[b]).wait()
            nxt = i + 1

            @pl.when(nxt < nch)
            def _():
                fire(1 - b, nxt)

            @pl.when(i >= 2)
            def _():
                pltpu.make_async_copy(
                    mlo.at[b], out_hbm.at[pl.ds(e0, CH1)], semo[b]).wait()
                pltpu.make_async_copy(
                    mhi.at[b], out_hbm.at[pl.ds(e0, CH1)], semo[b]).wait()

            @plsc.parallel_loop(0, CH1, unroll=2)
            def edge_body(j):
                acc1 = jnp.zeros((L,), jnp.float32)
                acc2 = jnp.zeros((L,), jnp.float32)
                xs = []
                for k in range(D // (2 * L)):
                    sl = pl.ds(k * L, L)
                    pb16 = plsc.bitcast(pbuf[b, j, sl], jnp.bfloat16)
                    qb16 = plsc.bitcast(qbuf[b, j, sl], jnp.bfloat16)
                    xb = pb16 + qb16
                    xe, xo = plsc.unpack(xb, format=plsc.PackFormat.INTERLEAVED)
                    xs.append(xe)
                    xs.append(xo)
                    acc1 = acc1 + xe + xo
                    acc2 = acc2 + xe * xe + xo * xo
                s1 = jnp.sum(acc1)
                s2 = jnp.sum(acc2)
                mu = s1 * (1.0 / D)
                var = s2 * (1.0 / D) - mu * mu
                rs = _rsqrt_v(jnp.full((L,), 1e-5, jnp.float32) + var)
                vmu = jnp.zeros((L,), jnp.float32) + mu
                one = jnp.full((L,), 1.0, jnp.float32)
                for k in range(D // (2 * L)):
                    sl = pl.ds(k * L, L)
                    mlo[b, j, sl] = (xs[2 * k] - vmu) * rs + one
                    mhi[b, j, sl] = (xs[2 * k + 1] - vmu) * rs + one

            pltpu.async_copy(mlo.at[b], out_hbm.at[pl.ds(base, CH1)], semo[b])
            pltpu.async_copy(mhi.at[b], out_hbm.at[pl.ds(EPAD + base, CH1)],
                             semo[b])
        return carry

    lax.fori_loop(0, nch // 2, pair_body, 0)
    for b in range(2):
        pltpu.make_async_copy(
            mlo.at[b], out_hbm.at[pl.ds(e0, CH1)], semo[b]).wait()
        pltpu.make_async_copy(
            mhi.at[b], out_hbm.at[pl.ds(e0, CH1)], semo[b]).wait()


@functools.partial(
    pl.kernel,
    out_type=jax.ShapeDtypeStruct((2 * NPAD, 128), jnp.float32),
    mesh=_sc_mesh,
    compiler_params=_sc_params,
    scratch_types=[
        pltpu.VMEM((2, CH2), jnp.int32),
        pltpu.VMEM((2, CH2, 128), jnp.float32),
        pltpu.VMEM_SHARED((NPAD, 128), jnp.float32),
        pltpu.SemaphoreType.DMA,
        pltpu.SemaphoreType.DMA,
    ],
)
def _scatter_kernel(m_hbm, r_hbm, out_hbm, ridx, chunk, acc, sm0, sm1):
    cid = lax.axis_index("c")
    sid = lax.axis_index("s")
    semm = [sm0, sm1]

    def zrow(j, c2):
        for k in range(128 // L):
            chunk[0, j, pl.ds(k * L, L)] = jnp.zeros((L,), jnp.float32)
        return c2

    lax.fori_loop(0, CH2, zrow, 0)
    for m in range(RPT // CH2):
        pltpu.sync_copy(chunk.at[0], acc.at[pl.ds(sid * RPT + m * CH2, CH2)])
    plsc.subcore_barrier()

    def fire(b, i):
        base = sid * ESC + i * CH2
        pltpu.sync_copy(r_hbm.at[pl.ds(base, CH2)], ridx.at[b])
        pltpu.async_copy(m_hbm.at[pl.ds(cid * EPAD + base, CH2)],
                         chunk.at[b], semm[b])

    fire(0, 0)

    def pair_body(i2, carry):
        for b in range(2):
            i = 2 * i2 + b
            pltpu.make_async_copy(
                m_hbm.at[pl.ds(cid * EPAD, CH2)], chunk.at[b],
                semm[b]).wait()
            nxt = i + 1

            @pl.when(nxt < NCH2)
            def _():
                fire(1 - b, nxt)

            pltpu.sync_copy(chunk.at[b], acc.at[ridx.at[b]], add=True)
        return carry

    lax.fori_loop(0, NCH2 // 2, pair_body, 0)
    plsc.subcore_barrier()
    rb = sid * RPT
    pltpu.sync_copy(acc.at[pl.ds(rb, RPT)],
                    out_hbm.at[pl.ds(cid * NPAD + rb, RPT)])


def _proj_body(x_ref, wt_ref, wb_ref, bm_ref, p_ref, q_ref):
    def pack_halves(v):
        lo = lax.bitcast_convert_type(
            v[:, :D // 2].astype(jnp.bfloat16), jnp.uint16).astype(jnp.uint32)
        hi = lax.bitcast_convert_type(
            v[:, D // 2:].astype(jnp.bfloat16), jnp.uint16).astype(jnp.uint32)
        return lo | (hi << 16)

    pv = (jnp.dot(x_ref[...], wt_ref[...],
                  preferred_element_type=jnp.float32) + bm_ref[...])
    qv = jnp.dot(x_ref[...], wb_ref[...], preferred_element_type=jnp.float32)
    p_ref[...] = pack_halves(pv)
    q_ref[...] = pack_halves(qv)


def _update_body(x_ref, lo_ref, hi_ref, w1_ref, w2a_ref, w2b_ref,
                 g1_ref, be1_ref, b_ref, g_ref, be_ref, o_ref):
    g1v = g1_ref[...]
    w2a = w2a_ref[...]
    w2b = w2b_ref[...]
    lo = lo_ref[...]
    hi = hi_ref[...]
    cnt = (jnp.sum(lo, axis=-1, keepdims=True)
           + jnp.sum(hi, axis=-1, keepdims=True)) * (1.0 / D)
    acc = jnp.dot(x_ref[...], w1_ref[...], preferred_element_type=jnp.float32)
    acc = acc + jnp.dot((lo - cnt) * g1v[0, :128], w2a,
                        preferred_element_type=jnp.float32)
    acc = acc + jnp.dot((hi - cnt) * g1v[0, 128:], w2b,
                        preferred_element_type=jnp.float32)
    be1v = be1_ref[...]
    bev = jnp.dot(be1v[:, :128], w2a, preferred_element_type=jnp.float32)
    bev = bev + jnp.dot(be1v[:, 128:], w2b, preferred_element_type=jnp.float32)
    acc = acc + b_ref[...] + cnt * bev
    mu = jnp.mean(acc, axis=-1, keepdims=True)
    var = jnp.mean((acc - mu) ** 2, axis=-1, keepdims=True)
    o_ref[...] = (acc - mu) * lax.rsqrt(var + 1e-5) * g_ref[...] + be_ref[...]


def kernel(nodes, senders, receivers, W_msg, b_msg, g1, be1,
           W_node, b_node, g2, be2):
    n = nodes.shape[1]
    e = senders.shape[0]
    x = jnp.pad(nodes[0], ((0, NPAD - n), (0, 0)))
    rp = jnp.concatenate(
        [receivers, jnp.full((EPAD - e,), n, jnp.int32)])
    sp = jnp.concatenate(
        [senders, jnp.zeros((EPAD - e,), jnp.int32)])

    grid = NPAD // MBLK
    p, q = pl.pallas_call(
        _proj_body,
        grid=(grid,),
        in_specs=[
            pl.BlockSpec((MBLK, D), lambda i: (i, 0)),
            pl.BlockSpec((D, D), lambda i: (0, 0)),
            pl.BlockSpec((D, D), lambda i: (0, 0)),
            pl.BlockSpec((1, D), lambda i: (0, 0)),
        ],
        out_specs=[
            pl.BlockSpec((MBLK, D // 2), lambda i: (i, 0)),
            pl.BlockSpec((MBLK, D // 2), lambda i: (i, 0)),
        ],
        out_shape=[
            jax.ShapeDtypeStruct((NPAD, D // 2), jnp.uint32),
            jax.ShapeDtypeStruct((NPAD, D // 2), jnp.uint32),
        ],
    )(x, W_msg[:D], W_msg[D:], b_msg[None])

    msgs = _msg_kernel(p, q, rp, sp)
    inbox2 = _scatter_kernel(msgs, rp)

    out = pl.pallas_call(
        _update_body,
        grid=(grid,),
        in_specs=[
            pl.BlockSpec((MBLK, D), lambda i: (i, 0)),
            pl.BlockSpec((MBLK, 128), lambda i: (i, 0)),
            pl.BlockSpec((MBLK, 128), lambda i: (i + NPAD // MBLK, 0)),
            pl.BlockSpec((D, D), lambda i: (0, 0)),
            pl.BlockSpec((128, D), lambda i: (0, 0)),
            pl.BlockSpec((128, D), lambda i: (0, 0)),
            pl.BlockSpec((1, D), lambda i: (0, 0)),
            pl.BlockSpec((1, D), lambda i: (0, 0)),
            pl.BlockSpec((1, D), lambda i: (0, 0)),
            pl.BlockSpec((1, D), lambda i: (0, 0)),
            pl.BlockSpec((1, D), lambda i: (0, 0)),
        ],
        out_specs=pl.BlockSpec((MBLK, D), lambda i: (i, 0)),
        out_shape=jax.ShapeDtypeStruct((NPAD, D), jnp.float32),
    )(x, inbox2, inbox2, W_node[:D], W_node[D:D + 128], W_node[D + 128:],
      g1[None], be1[None], b_node[None], g2[None], be2[None])
    return out[:n][None]


# speedup vs baseline: 1.0616x; 1.0518x over previous
import functools

import jax
import jax.numpy as jnp
from jax import lax
from jax.experimental import pallas as pl
from jax.experimental.pallas import tpu as pltpu
from jax.experimental.pallas import tpu_sc as plsc

D = 256
L = 16
NC, NS = 2, 16
NW = NC * NS
NPAD = 10240
EPAD = 163840
EW = EPAD // NW
EW0 = 6912
EW1 = EPAD // NS - EW0
CH1 = 64
CH2 = 128
ESC = EPAD // NS
NCH2 = ESC // CH2
RPT = NPAD // NS
MBLK = 1024


def _rsqrt_v(v):
    i = plsc.bitcast(v, jnp.int32)
    y = plsc.bitcast(jnp.int32(0x5F3759DF) - lax.shift_right_arithmetic(i, 1),
                     jnp.float32)
    for _ in range(3):
        y = y * (1.5 - 0.5 * v * y * y)
    return y


_sc_mesh = plsc.VectorSubcoreMesh(core_axis_name="c", subcore_axis_name="s")
_sc_params = pltpu.CompilerParams(needs_layout_passes=False)


@functools.partial(
    pl.kernel,
    out_type=jax.ShapeDtypeStruct((2 * EPAD, 128), jnp.float32),
    mesh=_sc_mesh,
    compiler_params=_sc_params,
    scratch_types=[
        pltpu.VMEM((EW0,), jnp.int32),
        pltpu.VMEM((EW0,), jnp.int32),
        pltpu.VMEM((2, CH1, D // 2), jnp.uint32),
        pltpu.VMEM((2, CH1, D // 2), jnp.uint32),
        pltpu.VMEM((2, CH1, 128), jnp.float32),
        pltpu.VMEM((2, CH1, 128), jnp.float32),
        pltpu.SemaphoreType.DMA,
        pltpu.SemaphoreType.DMA,
        pltpu.SemaphoreType.DMA,
        pltpu.SemaphoreType.DMA,
        pltpu.SemaphoreType.DMA,
        pltpu.SemaphoreType.DMA,
    ],
)
def _msg_kernel(p_hbm, q_hbm, r_hbm, s_hbm, out_hbm,
                ridx, sidx, pbuf, qbuf, mlo, mhi,
                sp0, sp1, sq0, sq1, so0, so1):
    cid = lax.axis_index("c")
    sid = lax.axis_index("s")
    is0 = cid == 0
    tcnt = jnp.where(is0, EW0, EW1)
    e0 = jnp.where(is0, sid * EW0, NS * EW0 + sid * EW1)
    nch = tcnt // CH1
    semp = [sp0, sp1]
    semq = [sq0, sq1]
    semo = [so0, so1]

    @pl.when(is0)
    def _():
        pltpu.sync_copy(r_hbm.at[pl.ds(e0, EW0)], ridx)
        pltpu.sync_copy(s_hbm.at[pl.ds(e0, EW0)], sidx)

    @pl.when(jnp.logical_not(is0))
    def _():
        pltpu.sync_copy(r_hbm.at[pl.ds(e0, EW1)], ridx.at[pl.ds(0, EW1)])
        pltpu.sync_copy(s_hbm.at[pl.ds(e0, EW1)], sidx.at[pl.ds(0, EW1)])

    def fire(b, i):
        pltpu.async_copy(p_hbm.at[ridx.at[pl.ds(i * CH1, CH1)]],
                         pbuf.at[b], semp[b])
        pltpu.async_copy(q_hbm.at[sidx.at[pl.ds(i * CH1, CH1)]],
                         qbuf.at[b], semq[b])

    fire(0, 0)

    def pair_body(i2, carry):
        for b in range(2):
            i = 2 * i2 + b
            base = e0 + i * CH1
            pltpu.make_async_copy(
                p_hbm.at[ridx.at[pl.ds(0, CH1)]], pbuf.at[b], semp[b]).wait()
            pltpu.make_async_copy(
                q_hbm.at[sidx.at[pl.ds(0, CH1)]], qbuf.at[b], semq[b]).wait()
            nxt = i + 1

            @pl.when(nxt < nch)
            def _():
                fire(1 - b, nxt)

            @pl.when(i >= 2)
            def _():
                pltpu.make_async_copy(
                    mlo.at[b], out_hbm.at[pl.ds(e0, CH1)], semo[b]).wait()
                pltpu.make_async_copy(
                    mhi.at[b], out_hbm.at[pl.ds(e0, CH1)], semo[b]).wait()

            @plsc.parallel_loop(0, CH1, unroll=2)
            def edge_body(j):
                acc1 = jnp.zeros((L,), jnp.float32)
                acc2 = jnp.zeros((L,), jnp.float32)
                xs = []
                for k in range(D // (2 * L)):
                    sl = pl.ds(k * L, L)
                    pb16 = plsc.bitcast(pbuf[b, j, sl], jnp.bfloat16)
                    qb16 = plsc.bitcast(qbuf[b, j, sl], jnp.bfloat16)
                    xb = pb16 + qb16
                    xe, xo = plsc.unpack(xb, format=plsc.PackFormat.INTERLEAVED)
                    xs.append(xe)
                    xs.append(xo)
                    acc1 = acc1 + xe + xo
                    acc2 = acc2 + xe * xe + xo * xo
                s1 = jnp.sum(acc1)
                s2 = jnp.sum(acc2)
                mu = s1 * (1.0 / D)
                var = s2 * (1.0 / D) - mu * mu
                rs = _rsqrt_v(jnp.full((L,), 1e-5, jnp.float32) + var)
                vmu = jnp.zeros((L,), jnp.float32) + mu
                one = jnp.full((L,), 1.0, jnp.float32)
                for k in range(D // (2 * L)):
                    sl = pl.ds(k * L, L)
                    mlo[b, j, sl] = (xs[2 * k] - vmu) * rs + one
                    mhi[b, j, sl] = (xs[2 * k + 1] - vmu) * rs + one

            pltpu.async_copy(mlo.at[b], out_hbm.at[pl.ds(base, CH1)], semo[b])
            pltpu.async_copy(mhi.at[b], out_hbm.at[pl.ds(EPAD + base, CH1)],
                             semo[b])
        return carry

    lax.fori_loop(0, nch // 2, pair_body, 0)
    for b in range(2):
        pltpu.make_async_copy(
            mlo.at[b], out_hbm.at[pl.ds(e0, CH1)], semo[b]).wait()
        pltpu.make_async_copy(
            mhi.at[b], out_hbm.at[pl.ds(e0, CH1)], semo[b]).wait()


@functools.partial(
    pl.kernel,
    out_type=jax.ShapeDtypeStruct((2 * NPAD, 128), jnp.float32),
    mesh=_sc_mesh,
    compiler_params=_sc_params,
    scratch_types=[
        pltpu.VMEM((2, CH2), jnp.int32),
        pltpu.VMEM((2, CH2, 128), jnp.float32),
        pltpu.VMEM_SHARED((NPAD, 128), jnp.float32),
        pltpu.SemaphoreType.DMA,
        pltpu.SemaphoreType.DMA,
    ],
)
def _scatter_kernel(m_hbm, r_hbm, out_hbm, ridx, chunk, acc, sm0, sm1):
    cid = lax.axis_index("c")
    sid = lax.axis_index("s")
    semm = [sm0, sm1]

    def zrow(j, c2):
        for k in range(128 // L):
            chunk[0, j, pl.ds(k * L, L)] = jnp.zeros((L,), jnp.float32)
        return c2

    lax.fori_loop(0, CH2, zrow, 0)
    for m in range(RPT // CH2):
        pltpu.sync_copy(chunk.at[0], acc.at[pl.ds(sid * RPT + m * CH2, CH2)])
    plsc.subcore_barrier()

    def fire(b, i):
        base = sid * ESC + i * CH2
        pltpu.sync_copy(r_hbm.at[pl.ds(base, CH2)], ridx.at[b])
        pltpu.async_copy(m_hbm.at[pl.ds(cid * EPAD + base, CH2)],
                         chunk.at[b], semm[b])

    fire(0, 0)

    def pair_body(i2, carry):
        for b in range(2):
            i = 2 * i2 + b
            pltpu.make_async_copy(
                m_hbm.at[pl.ds(cid * EPAD, CH2)], chunk.at[b],
                semm[b]).wait()
            nxt = i + 1

            @pl.when(nxt < NCH2)
            def _():
                fire(1 - b, nxt)

            pltpu.sync_copy(chunk.at[b], acc.at[ridx.at[b]], add=True)
        return carry

    lax.fori_loop(0, NCH2 // 2, pair_body, 0)
    plsc.subcore_barrier()
    rb = sid * RPT
    pltpu.sync_copy(acc.at[pl.ds(rb, RPT)],
                    out_hbm.at[pl.ds(cid * NPAD + rb, RPT)])


def _proj_body(x_ref, wt_ref, wb_ref, bm_ref, p_ref, q_ref):
    def pack_halves(v):
        lo = lax.bitcast_convert_type(
            v[:, :D // 2].astype(jnp.bfloat16), jnp.uint16).astype(jnp.uint32)
        hi = lax.bitcast_convert_type(
            v[:, D // 2:].astype(jnp.bfloat16), jnp.uint16).astype(jnp.uint32)
        return lo | (hi << 16)

    pv = (jnp.dot(x_ref[...], wt_ref[...],
                  preferred_element_type=jnp.float32) + bm_ref[...])
    qv = jnp.dot(x_ref[...], wb_ref[...], preferred_element_type=jnp.float32)
    p_ref[...] = pack_halves(pv)
    q_ref[...] = pack_halves(qv)


def _update_body(x_ref, lo_ref, hi_ref, w1_ref, w2a_ref, w2b_ref,
                 g1_ref, be1_ref, b_ref, g_ref, be_ref, o_ref):
    g1v = g1_ref[...]
    w2a = w2a_ref[...]
    w2b = w2b_ref[...]
    lo = lo_ref[...]
    hi = hi_ref[...]
    cnt = (jnp.sum(lo, axis=-1, keepdims=True)
           + jnp.sum(hi, axis=-1, keepdims=True)) * (1.0 / D)
    acc = jnp.dot(x_ref[...], w1_ref[...], preferred_element_type=jnp.float32)
    acc = acc + jnp.dot((lo - cnt) * g1v[0, :128], w2a,
                        preferred_element_type=jnp.float32)
    acc = acc + jnp.dot((hi - cnt) * g1v[0, 128:], w2b,
                        preferred_element_type=jnp.float32)
    be1v = be1_ref[...]
    bev = jnp.dot(be1v[:, :128], w2a, preferred_element_type=jnp.float32)
    bev = bev + jnp.dot(be1v[:, 128:], w2b, preferred_element_type=jnp.float32)
    acc = acc + b_ref[...] + cnt * bev
    mu = jnp.mean(acc, axis=-1, keepdims=True)
    var = jnp.mean((acc - mu) ** 2, axis=-1, keepdims=True)
    o_ref[...] = (acc - mu) * lax.rsqrt(var + 1e-5) * g_ref[...] + be_ref[...]


def kernel(nodes, senders, receivers, W_msg, b_msg, g1, be1,
           W_node, b_node, g2, be2):
    n = nodes.shape[1]
    e = senders.shape[0]
    x = jnp.pad(nodes[0], ((0, NPAD - n), (0, 0)))
    rp = jnp.concatenate(
        [receivers, jnp.full((EPAD - e,), n, jnp.int32)])
    sp = jnp.concatenate(
        [senders, jnp.zeros((EPAD - e,), jnp.int32)])

    grid = NPAD // MBLK
    p, q = pl.pallas_call(
        _proj_body,
        grid=(grid,),
        in_specs=[
            pl.BlockSpec((MBLK, D), lambda i: (i, 0)),
            pl.BlockSpec((D, D), lambda i: (0, 0)),
            pl.BlockSpec((D, D), lambda i: (0, 0)),
            pl.BlockSpec((1, D), lambda i: (0, 0)),
        ],
        out_specs=[
            pl.BlockSpec((MBLK, D // 2), lambda i: (i, 0)),
            pl.BlockSpec((MBLK, D // 2), lambda i: (i, 0)),
        ],
        out_shape=[
            jax.ShapeDtypeStruct((NPAD, D // 2), jnp.uint32),
            jax.ShapeDtypeStruct((NPAD, D // 2), jnp.uint32),
        ],
    )(x, W_msg[:D], W_msg[D:], b_msg[None])

    msgs = _msg_kernel(p, q, rp, sp)
    inbox2 = _scatter_kernel(msgs, rp)

    out = pl.pallas_call(
        _update_body,
        grid=(grid,),
        in_specs=[
            pl.BlockSpec((MBLK, D), lambda i: (i, 0)),
            pl.BlockSpec((MBLK, 128), lambda i: (i, 0)),
            pl.BlockSpec((MBLK, 128), lambda i: (i + NPAD // MBLK, 0)),
            pl.BlockSpec((D, D), lambda i: (0, 0)),
            pl.BlockSpec((128, D), lambda i: (0, 0)),
            pl.BlockSpec((128, D), lambda i: (0, 0)),
            pl.BlockSpec((1, D), lambda i: (0, 0)),
            pl.BlockSpec((1, D), lambda i: (0, 0)),
            pl.BlockSpec((1, D), lambda i: (0, 0)),
            pl.BlockSpec((1, D), lambda i: (0, 0)),
            pl.BlockSpec((1, D), lambda i: (0, 0)),
        ],
        out_specs=pl.BlockSpec((MBLK, D), lambda i: (i, 0)),
        out_shape=jax.ShapeDtypeStruct((NPAD, D), jnp.float32),
    )(x, inbox2, inbox2, W_node[:D], W_node[D:D + 128], W_node[D + 128:],
      g1[None], be1[None], b_node[None], g2[None], be2[None])
    return out[:n][None]


# speedup vs baseline: 1.1276x; 1.0622x over previous
import functools

import jax
import jax.numpy as jnp
from jax import lax
from jax.experimental import pallas as pl
from jax.experimental.pallas import tpu as pltpu
from jax.experimental.pallas import tpu_sc as plsc

D = 256
L = 16
NC, NS = 2, 16
NW = NC * NS
NPAD = 10240
EPAD = 163840
EW = EPAD // NW
EW0 = 6912
EW1 = EPAD // NS - EW0
CH1 = 64
CH2 = 128
ESC = EPAD // NS
NCH2 = ESC // CH2
RPT = NPAD // NS
MBLK = 1024


def _rsqrt_v(v):
    i = plsc.bitcast(v, jnp.int32)
    y = plsc.bitcast(jnp.int32(0x5F3759DF) - lax.shift_right_arithmetic(i, 1),
                     jnp.float32)
    for _ in range(3):
        y = y * (1.5 - 0.5 * v * y * y)
    return y


_sc_mesh = plsc.VectorSubcoreMesh(core_axis_name="c", subcore_axis_name="s")
_sc_params = pltpu.CompilerParams(needs_layout_passes=False)


@functools.partial(
    pl.kernel,
    out_type=jax.ShapeDtypeStruct((2 * EPAD, 128), jnp.float32),
    mesh=_sc_mesh,
    compiler_params=_sc_params,
    scratch_types=[
        pltpu.VMEM((EW0,), jnp.int32),
        pltpu.VMEM((EW0,), jnp.int32),
        pltpu.VMEM((2, CH1, D // 2), jnp.uint32),
        pltpu.VMEM((2, CH1, D // 2), jnp.uint32),
        pltpu.VMEM((2, CH1, 128), jnp.float32),
        pltpu.VMEM((2, CH1, 128), jnp.float32),
        pltpu.SemaphoreType.DMA,
        pltpu.SemaphoreType.DMA,
        pltpu.SemaphoreType.DMA,
        pltpu.SemaphoreType.DMA,
        pltpu.SemaphoreType.DMA,
        pltpu.SemaphoreType.DMA,
    ],
)
def _msg_kernel(p_hbm, q_hbm, r_hbm, s_hbm, out_hbm,
                ridx, sidx, pbuf, qbuf, mlo, mhi,
                sp0, sp1, sq0, sq1, so0, so1):
    cid = lax.axis_index("c")
    sid = lax.axis_index("s")
    is0 = cid == 0
    tcnt = jnp.where(is0, EW0, EW1)
    e0 = jnp.where(is0, sid * EW0, NS * EW0 + sid * EW1)
    nch = tcnt // CH1
    semp = [sp0, sp1]
    semq = [sq0, sq1]
    semo = [so0, so1]

    @pl.when(is0)
    def _():
        pltpu.sync_copy(r_hbm.at[pl.ds(e0, EW0)], ridx)
        pltpu.sync_copy(s_hbm.at[pl.ds(e0, EW0)], sidx)

    @pl.when(jnp.logical_not(is0))
    def _():
        pltpu.sync_copy(r_hbm.at[pl.ds(e0, EW1)], ridx.at[pl.ds(0, EW1)])
        pltpu.sync_copy(s_hbm.at[pl.ds(e0, EW1)], sidx.at[pl.ds(0, EW1)])

    def fire(b, i):
        pltpu.async_copy(p_hbm.at[ridx.at[pl.ds(i * CH1, CH1)]],
                         pbuf.at[b], semp[b])
        pltpu.async_copy(q_hbm.at[sidx.at[pl.ds(i * CH1, CH1)]],
                         qbuf.at[b], semq[b])

    fire(0, 0)

    def pair_body(i2, carry):
        for b in range(2):
            i = 2 * i2 + b
            base = e0 + i * CH1
            pltpu.make_async_copy(
                p_hbm.at[ridx.at[pl.ds(0, CH1)]], pbuf.at[b], semp[b]).wait()
            pltpu.make_async_copy(
                q_hbm.at[sidx.at[pl.ds(0, CH1)]], qbuf.at[b], semq[b]).wait()
            nxt = i + 1

            @pl.when(nxt < nch)
            def _():
                fire(1 - b, nxt)

            @pl.when(i >= 2)
            def _():
                pltpu.make_async_copy(
                    mlo.at[b], out_hbm.at[pl.ds(e0, CH1)], semo[b]).wait()
                pltpu.make_async_copy(
                    mhi.at[b], out_hbm.at[pl.ds(e0, CH1)], semo[b]).wait()

            @plsc.parallel_loop(0, CH1, unroll=2)
            def edge_body(j):
                acc1 = jnp.zeros((L,), jnp.float32)
                acc2 = jnp.zeros((L,), jnp.float32)
                xs = []
                for k in range(D // (2 * L)):
                    sl = pl.ds(k * L, L)
                    pb16 = plsc.bitcast(pbuf[b, j, sl], jnp.bfloat16)
                    qb16 = plsc.bitcast(qbuf[b, j, sl], jnp.bfloat16)
                    xb = pb16 + qb16
                    xe, xo = plsc.unpack(xb, format=plsc.PackFormat.INTERLEAVED)
                    xs.append(xe)
                    xs.append(xo)
                    acc1 = acc1 + xe + xo
                    acc2 = acc2 + xe * xe + xo * xo
                s1 = jnp.sum(acc1)
                s2 = jnp.sum(acc2)
                mu = s1 * (1.0 / D)
                var = s2 * (1.0 / D) - mu * mu
                rs = _rsqrt_v(jnp.full((L,), 1e-5, jnp.float32) + var)
                vmu = jnp.zeros((L,), jnp.float32) + mu
                one = jnp.full((L,), 1.0, jnp.float32)
                for k in range(D // (2 * L)):
                    sl = pl.ds(k * L, L)
                    mlo[b, j, sl] = (xs[2 * k] - vmu) * rs + one
                    mhi[b, j, sl] = (xs[2 * k + 1] - vmu) * rs + one

            pltpu.async_copy(mlo.at[b], out_hbm.at[pl.ds(base, CH1)], semo[b])
            pltpu.async_copy(mhi.at[b], out_hbm.at[pl.ds(EPAD + base, CH1)],
                             semo[b])
        return carry

    lax.fori_loop(0, nch // 2, pair_body, 0)
    for b in range(2):
        pltpu.make_async_copy(
            mlo.at[b], out_hbm.at[pl.ds(e0, CH1)], semo[b]).wait()
        pltpu.make_async_copy(
            mhi.at[b], out_hbm.at[pl.ds(e0, CH1)], semo[b]).wait()


@functools.partial(
    pl.kernel,
    out_type=jax.ShapeDtypeStruct((2 * NPAD, 128), jnp.float32),
    mesh=_sc_mesh,
    compiler_params=_sc_params,
    scratch_types=[
        pltpu.VMEM((NCH2, CH2), jnp.int32),
        pltpu.VMEM((2, CH2, 128), jnp.float32),
        pltpu.VMEM_SHARED((NPAD, 128), jnp.float32),
        pltpu.SemaphoreType.DMA,
        pltpu.SemaphoreType.DMA,
        pltpu.SemaphoreType.DMA,
        pltpu.SemaphoreType.DMA,
    ],
)
def _scatter_kernel(m_hbm, r2_hbm, out_hbm, ridx, chunk, acc,
                    sm0, sm1, ss0, ss1):
    cid = lax.axis_index("c")
    sid = lax.axis_index("s")
    semm = [sm0, sm1]
    sems = [ss0, ss1]

    def zrow(j, c2):
        for k in range(128 // L):
            chunk[0, j, pl.ds(k * L, L)] = jnp.zeros((L,), jnp.float32)
        return c2

    lax.fori_loop(0, CH2, zrow, 0)
    for m in range(RPT // CH2):
        pltpu.sync_copy(chunk.at[0], acc.at[pl.ds(sid * RPT + m * CH2, CH2)])
    pltpu.sync_copy(r2_hbm.at[sid], ridx)
    plsc.subcore_barrier()

    def fire(b, i):
        base = sid * ESC + i * CH2
        pltpu.async_copy(m_hbm.at[pl.ds(cid * EPAD + base, CH2)],
                         chunk.at[b], semm[b])

    fire(0, 0)

    def pair_body(i2, carry):
        for b in range(2):
            i = 2 * i2 + b
            pltpu.make_async_copy(
                m_hbm.at[pl.ds(cid * EPAD, CH2)], chunk.at[b],
                semm[b]).wait()
            nxt = i + 1

            @pl.when(nxt < NCH2)
            def _():
                fire(1 - b, nxt)

            @pl.when(i >= 2)
            def _():
                pltpu.make_async_copy(
                    chunk.at[b], acc.at[ridx.at[0]], sems[b]).wait()

            pltpu.async_copy(chunk.at[b], acc.at[ridx.at[i]], sems[b],
                             add=True)
        return carry

    lax.fori_loop(0, NCH2 // 2, pair_body, 0)
    for b in range(2):
        pltpu.make_async_copy(
            chunk.at[b], acc.at[ridx.at[0]], sems[b]).wait()
    plsc.subcore_barrier()
    rb = sid * RPT
    pltpu.sync_copy(acc.at[pl.ds(rb, RPT)],
                    out_hbm.at[pl.ds(cid * NPAD + rb, RPT)])


def _proj_body(x_ref, wt_ref, wb_ref, bm_ref, p_ref, q_ref):
    def pack_halves(v):
        lo = lax.bitcast_convert_type(
            v[:, :D // 2].astype(jnp.bfloat16), jnp.uint16).astype(jnp.uint32)
        hi = lax.bitcast_convert_type(
            v[:, D // 2:].astype(jnp.bfloat16), jnp.uint16).astype(jnp.uint32)
        return lo | (hi << 16)

    pv = (jnp.dot(x_ref[...], wt_ref[...],
                  preferred_element_type=jnp.float32) + bm_ref[...])
    qv = jnp.dot(x_ref[...], wb_ref[...], preferred_element_type=jnp.float32)
    p_ref[...] = pack_halves(pv)
    q_ref[...] = pack_halves(qv)


def _update_body(x_ref, lo_ref, hi_ref, w1_ref, w2a_ref, w2b_ref,
                 g1_ref, be1_ref, b_ref, g_ref, be_ref, o_ref):
    g1v = g1_ref[...]
    w2a = w2a_ref[...]
    w2b = w2b_ref[...]
    lo = lo_ref[...]
    hi = hi_ref[...]
    cnt = (jnp.sum(lo, axis=-1, keepdims=True)
           + jnp.sum(hi, axis=-1, keepdims=True)) * (1.0 / D)
    acc = jnp.dot(x_ref[...], w1_ref[...], preferred_element_type=jnp.float32)
    acc = acc + jnp.dot((lo - cnt) * g1v[0, :128], w2a,
                        preferred_element_type=jnp.float32)
    acc = acc + jnp.dot((hi - cnt) * g1v[0, 128:], w2b,
                        preferred_element_type=jnp.float32)
    be1v = be1_ref[...]
    bev = jnp.dot(be1v[:, :128], w2a, preferred_element_type=jnp.float32)
    bev = bev + jnp.dot(be1v[:, 128:], w2b, preferred_element_type=jnp.float32)
    acc = acc + b_ref[...] + cnt * bev
    mu = jnp.mean(acc, axis=-1, keepdims=True)
    var = jnp.mean((acc - mu) ** 2, axis=-1, keepdims=True)
    o_ref[...] = (acc - mu) * lax.rsqrt(var + 1e-5) * g_ref[...] + be_ref[...]


def kernel(nodes, senders, receivers, W_msg, b_msg, g1, be1,
           W_node, b_node, g2, be2):
    n = nodes.shape[1]
    e = senders.shape[0]
    x = jnp.pad(nodes[0], ((0, NPAD - n), (0, 0)))
    rp = jnp.concatenate(
        [receivers, jnp.full((EPAD - e,), n, jnp.int32)])
    sp = jnp.concatenate(
        [senders, jnp.zeros((EPAD - e,), jnp.int32)])

    grid = NPAD // MBLK
    p, q = pl.pallas_call(
        _proj_body,
        grid=(grid,),
        in_specs=[
            pl.BlockSpec((MBLK, D), lambda i: (i, 0)),
            pl.BlockSpec((D, D), lambda i: (0, 0)),
            pl.BlockSpec((D, D), lambda i: (0, 0)),
            pl.BlockSpec((1, D), lambda i: (0, 0)),
        ],
        out_specs=[
            pl.BlockSpec((MBLK, D // 2), lambda i: (i, 0)),
            pl.BlockSpec((MBLK, D // 2), lambda i: (i, 0)),
        ],
        out_shape=[
            jax.ShapeDtypeStruct((NPAD, D // 2), jnp.uint32),
            jax.ShapeDtypeStruct((NPAD, D // 2), jnp.uint32),
        ],
    )(x, W_msg[:D], W_msg[D:], b_msg[None])

    msgs = _msg_kernel(p, q, rp, sp)
    inbox2 = _scatter_kernel(msgs, rp.reshape(NS, NCH2, CH2))

    out = pl.pallas_call(
        _update_body,
        grid=(grid,),
        in_specs=[
            pl.BlockSpec((MBLK, D), lambda i: (i, 0)),
            pl.BlockSpec((MBLK, 128), lambda i: (i, 0)),
            pl.BlockSpec((MBLK, 128), lambda i: (i + NPAD // MBLK, 0)),
            pl.BlockSpec((D, D), lambda i: (0, 0)),
            pl.BlockSpec((128, D), lambda i: (0, 0)),
            pl.BlockSpec((128, D), lambda i: (0, 0)),
            pl.BlockSpec((1, D), lambda i: (0, 0)),
            pl.BlockSpec((1, D), lambda i: (0, 0)),
            pl.BlockSpec((1, D), lambda i: (0, 0)),
            pl.BlockSpec((1, D), lambda i: (0, 0)),
            pl.BlockSpec((1, D), lambda i: (0, 0)),
        ],
        out_specs=pl.BlockSpec((MBLK, D), lambda i: (i, 0)),
        out_shape=jax.ShapeDtypeStruct((NPAD, D), jnp.float32),
    )(x, inbox2, inbox2, W_node[:D], W_node[D:D + 128], W_node[D + 128:],
      g1[None], be1[None], b_node[None], g2[None], be2[None])
    return out[:n][None]
